# Initial kernel scaffold; baseline (speedup 1.0000x reference)
import jax
import jax.numpy as jnp
from jax.experimental import pallas as pl


def kernel(x, edge_index, W, a):
    raise NotImplementedError("write your pallas kernel here")


# speedup vs baseline: 2.2653x; 2.2653x over previous
import functools

import jax
import jax.numpy as jnp
from jax import lax
from jax.experimental import pallas as pl
from jax.experimental.pallas import tpu as pltpu
from jax.experimental.pallas import tpu_sc as plsc

N = 10000
E = 160000
D = 256
L = 16
CH = 128
NPAD = 10240

PER32 = 5120
PER16 = 10240

_mesh2 = plsc.VectorSubcoreMesh(core_axis_name="c", subcore_axis_name="s")
_mesh1 = plsc.VectorSubcoreMesh(core_axis_name="c", subcore_axis_name="s",
                                num_cores=1)


def _wid2():
    return lax.axis_index("s") * 2 + lax.axis_index("c")


def _wid1():
    return lax.axis_index("s")


def _tile_range(wid, per, last_tile, last_count):
    base = wid * per
    nch = jnp.where(wid == last_tile, last_count // CH, per // CH)
    return base, nch


def _matmul_body(x_ref, w_ref, o_ref):
    o_ref[...] = jnp.dot(x_ref[...], w_ref[...],
                         preferred_element_type=jnp.float32,
                         precision=lax.Precision.HIGHEST)


def _matmul(x2d, W):
    bm = 1000
    return pl.pallas_call(
        _matmul_body,
        grid=(N // bm,),
        in_specs=[
            pl.BlockSpec((bm, D), lambda i: (i, 0)),
            pl.BlockSpec((D, D), lambda i: (0, 0)),
        ],
        out_specs=pl.BlockSpec((bm, D), lambda i: (i, 0)),
        out_shape=jax.ShapeDtypeStruct((N, D), jnp.float32),
    )(x2d, W)


@functools.partial(
    pl.kernel,
    out_type=jax.ShapeDtypeStruct((E,), jnp.float32),
    mesh=_mesh2,
    compiler_params=pltpu.CompilerParams(use_tc_tiling_on_sc=False,
                                         needs_layout_passes=False),
    scratch_types=[
        pltpu.VMEM((CH,), jnp.int32),
        pltpu.VMEM((CH,), jnp.int32),
        pltpu.VMEM((CH, D), jnp.float32),
        pltpu.VMEM((CH, D), jnp.float32),
        pltpu.VMEM((D,), jnp.float32),
        pltpu.VMEM((CH,), jnp.float32),
        pltpu.SemaphoreType.DMA,
        pltpu.SemaphoreType.DMA,
    ],
)
def _edge_scores(h_hbm, e0_hbm, e1_hbm, a_hbm, out_hbm,
                 idx0, idx1, rows0, rows1, av, outv, sem0, sem1):
    base, nch = _tile_range(_wid2(), PER32, 31, 1280)
    pltpu.sync_copy(a_hbm, av)

    def chunk(i, _):
        b = base + i * CH
        pltpu.sync_copy(e0_hbm.at[pl.ds(b, CH)], idx0)
        pltpu.sync_copy(e1_hbm.at[pl.ds(b, CH)], idx1)
        cp0 = pltpu.async_copy(h_hbm.at[idx0], rows0, sem0)
        cp1 = pltpu.async_copy(h_hbm.at[idx1], rows1, sem1)
        cp0.wait()
        cp1.wait()

        def group(g, _):
            rid = g * L + lax.iota(jnp.int32, L)
            acc = jnp.zeros((L,), jnp.float32)
            for jd in range(D // L):
                avv = av[pl.ds(jd * L, L)]
                for t in range(L):
                    cid = jnp.full((L,), jd * L + t, jnp.int32)
                    r0 = plsc.load_gather(rows0, [rid, cid])
                    r1 = plsc.load_gather(rows1, [rid, cid])
                    acc = acc + avv[t] * jnp.abs(r0 - r1)
            outv[pl.ds(g * L, L)] = jnp.maximum(acc, 0.0)
            return 0

        lax.fori_loop(0, CH // L, group, 0)
        pltpu.sync_copy(outv, out_hbm.at[pl.ds(b, CH)])
        return 0

    lax.fori_loop(0, nch, chunk, 0)


@functools.partial(
    pl.kernel,
    out_type=jax.ShapeDtypeStruct((N * N,), jnp.int32),
    mesh=_mesh2,
    scratch_types=[
        pltpu.VMEM((1, CH), jnp.int32),
        pltpu.VMEM((CH,), jnp.int32),
        pltpu.VMEM((CH,), jnp.int32),
        pltpu.SemaphoreType.DMA,
    ],
)
def _scatter_ids(e0_hbm, e1_hbm, idbuf_hbm, keys2, tmp, ids, sem):
    base, nch = _tile_range(_wid2(), PER32, 31, 1280)

    def chunk(i, _):
        b = base + i * CH
        pltpu.sync_copy(e0_hbm.at[pl.ds(b, CH)], tmp)
        pltpu.sync_copy(e1_hbm.at[pl.ds(b, CH)], keys2.at[0])
        for j in range(CH // L):
            sl = pl.ds(j * L, L)
            keys2[0, sl] = tmp[sl] * N + keys2[0, sl]
            ids[sl] = (b + j * L) + lax.iota(jnp.int32, L)
        pltpu.async_copy(ids, idbuf_hbm.at[keys2.at[0]], sem).wait()
        return 0

    lax.fori_loop(0, nch, chunk, 0)


@functools.partial(
    pl.kernel,
    out_type=(jax.ShapeDtypeStruct((E,), jnp.float32),
              jax.ShapeDtypeStruct((E,), jnp.int32)),
    mesh=_mesh1,
    scratch_types=[
        pltpu.VMEM((1, CH), jnp.int32),
        pltpu.VMEM((CH,), jnp.int32),
        pltpu.VMEM((1, CH), jnp.int32),
        pltpu.VMEM((CH,), jnp.float32),
        pltpu.VMEM((2000,), jnp.float32),
        pltpu.VMEM_SHARED((E,), jnp.float32),
        pltpu.SemaphoreType.DMA,
    ],
)
def _coalesce(e0_hbm, e1_hbm, idbuf_hbm, v_hbm, acc_hbm, w_hbm,
              keys2, tmp, w2, vv, zbuf, acc_sp, sem):
    wid = _wid1()
    base, nch = _tile_range(wid, PER16, 15, 6400)

    def zfill(i, _):
        zbuf[pl.ds(i * L, L)] = jnp.zeros((L,), jnp.float32)
        return 0

    lax.fori_loop(0, 2000 // L, zfill, 0)
    for z in range(5):
        pltpu.sync_copy(zbuf, acc_sp.at[pl.ds(wid * 10000 + z * 2000, 2000)])
    plsc.subcore_barrier()

    def chunk(i, _):
        b = base + i * CH
        pltpu.sync_copy(e0_hbm.at[pl.ds(b, CH)], tmp)
        pltpu.sync_copy(e1_hbm.at[pl.ds(b, CH)], keys2.at[0])
        for j in range(CH // L):
            sl = pl.ds(j * L, L)
            keys2[0, sl] = tmp[sl] * N + keys2[0, sl]
        pltpu.async_copy(idbuf_hbm.at[keys2.at[0]], w2.at[0], sem).wait()
        pltpu.sync_copy(w2.at[0], w_hbm.at[pl.ds(b, CH)])
        pltpu.sync_copy(v_hbm.at[pl.ds(b, CH)], vv)
        pltpu.sync_copy(vv, acc_sp.at[w2.at[0]], add=True)
        return 0

    lax.fori_loop(0, nch, chunk, 0)
    plsc.subcore_barrier()
    for z in range(5):
        sl = pl.ds(wid * 10000 + z * 2000, 2000)
        pltpu.sync_copy(acc_sp.at[sl], zbuf)
        pltpu.sync_copy(zbuf, acc_hbm.at[sl])


@functools.partial(
    pl.kernel,
    out_type=(jax.ShapeDtypeStruct((E,), jnp.float32),
              jax.ShapeDtypeStruct((NPAD,), jnp.float32)),
    mesh=_mesh1,
    scratch_types=[
        pltpu.VMEM((1, CH), jnp.int32),
        pltpu.VMEM((1, CH), jnp.int32),
        pltpu.VMEM((CH,), jnp.float32),
        pltpu.VMEM((CH,), jnp.float32),
        pltpu.VMEM((640,), jnp.float32),
        pltpu.VMEM_SHARED((NPAD,), jnp.float32),
        pltpu.SemaphoreType.DMA,
    ],
)
def _col_sums(w_hbm, acc_hbm, e1_hbm, p_hbm, colsum_hbm,
              w2, col2, gv, pv, zbuf, cs_sp, sem):
    wid = _wid1()
    base, nch = _tile_range(wid, PER16, 15, 6400)

    def zfill(i, _):
        zbuf[pl.ds(i * L, L)] = jnp.zeros((L,), jnp.float32)
        return 0

    lax.fori_loop(0, 640 // L, zfill, 0)
    pltpu.sync_copy(zbuf, cs_sp.at[pl.ds(wid * 640, 640)])
    plsc.subcore_barrier()

    def chunk(i, _):
        b = base + i * CH
        pltpu.sync_copy(w_hbm.at[pl.ds(b, CH)], w2.at[0])
        pltpu.sync_copy(e1_hbm.at[pl.ds(b, CH)], col2.at[0])
        pltpu.async_copy(acc_hbm.at[w2.at[0]], gv, sem).wait()
        for j in range(CH // L):
            sl = pl.ds(j * L, L)
            ids = (b + j * L) + lax.iota(jnp.int32, L)
            p = jnp.exp(gv[sl])
            pv[sl] = p
            gv[sl] = jnp.where(w2[0, sl] == ids, p, 0.0)
        pltpu.sync_copy(pv, p_hbm.at[pl.ds(b, CH)])
        pltpu.sync_copy(gv, cs_sp.at[col2.at[0]], add=True)
        return 0

    lax.fori_loop(0, nch, chunk, 0)
    plsc.subcore_barrier()
    sl = pl.ds(wid * 640, 640)
    pltpu.sync_copy(cs_sp.at[sl], zbuf)
    pltpu.sync_copy(zbuf, colsum_hbm.at[sl])


ZCH = 50000


@functools.partial(
    pl.kernel,
    out_type=jax.ShapeDtypeStruct((N * N,), jnp.float32),
    mesh=_mesh1,
    scratch_types=[
        pltpu.VMEM((1, CH), jnp.int32),
        pltpu.VMEM((CH,), jnp.int32),
        pltpu.VMEM((CH,), jnp.int32),
        pltpu.VMEM((CH,), jnp.float32),
        pltpu.VMEM((CH,), jnp.float32),
        pltpu.VMEM((CH,), jnp.float32),
        pltpu.VMEM((ZCH,), jnp.float32),
        pltpu.SemaphoreType.DMA,
    ],
)
def _final_scatter(e0_hbm, e1_hbm, p_hbm, colsum_hbm, dense_hbm,
                   keys2, tmp, col, pv, sv, outv, zbuf, sem):
    wid = _wid1()
    base, nch = _tile_range(wid, PER16, 15, 6400)

    def zfill(i, _):
        zbuf[pl.ds(i * L, L)] = jnp.zeros((L,), jnp.float32)
        return 0

    lax.fori_loop(0, ZCH // L, zfill, 0)
    slab = wid * (N * N // 16)

    def zdma(i, _):
        pltpu.sync_copy(zbuf, dense_hbm.at[pl.ds(slab + i * ZCH, ZCH)])
        return 0

    lax.fori_loop(0, N * N // 16 // ZCH, zdma, 0)
    plsc.subcore_barrier()

    def chunk(i, _):
        b = base + i * CH
        pltpu.sync_copy(e0_hbm.at[pl.ds(b, CH)], tmp)
        pltpu.sync_copy(e1_hbm.at[pl.ds(b, CH)], col)
        pltpu.sync_copy(p_hbm.at[pl.ds(b, CH)], pv)
        pltpu.async_copy(colsum_hbm.at[col], sv, sem).wait()
        for j in range(CH // L):
            sl = pl.ds(j * L, L)
            keys2[0, sl] = tmp[sl] * N + col[sl]
            outv[sl] = pv[sl] / sv[sl]
        pltpu.async_copy(outv, dense_hbm.at[keys2.at[0]], sem).wait()
        return 0

    lax.fori_loop(0, nch, chunk, 0)


def kernel(x, edge_index, W, a):
    x2d = x[0]
    e0 = edge_index[0]
    e1 = edge_index[1]
    h2d = _matmul(x2d, W)
    edge_v = _edge_scores(h2d, e0, e1, a)
    idbuf = _scatter_ids(e0, e1)
    acc, w = _coalesce(e0, e1, idbuf, edge_v)
    p, colsum = _col_sums(w, acc, e1)
    dense = _final_scatter(e0, e1, p, colsum)
    return h2d[None], dense.reshape(1, N, N)
